# Initial kernel scaffold; baseline (speedup 1.0000x reference)
import jax
import jax.numpy as jnp
from jax.experimental import pallas as pl


def kernel(x, edge_indices, W, src_attn, dst_attn):
    raise NotImplementedError("write your pallas kernel here")


# speedup vs baseline: 54.0120x; 54.0120x over previous
import jax
import jax.numpy as jnp
from jax import lax
from jax.experimental import pallas as pl
from jax.experimental.pallas import tpu as pltpu, tpu_sc as plsc

N = 10000
E = 320000
D = 128
H = 8
DH = 16

NPAD = 10240
NC = 2
NS = 16
NW = NC * NS
K = 64
T_PER_W = 162
E_PAD = NW * K * T_PER_W
RPT = NPAD // NS
FBLK = 1024
CBLK = 1000

_GDN = lax.GatherDimensionNumbers(
    offset_dims=(), collapsed_slice_dims=(0,), start_index_map=(0,))


def _rot8(v):
    shift = jnp.bitwise_and(lax.iota(jnp.int32, 16) + 8, 15)
    return lax.gather(v, shift[:, None], _GDN, (1,),
                      mode=lax.GatherScatterMode.PROMISE_IN_BOUNDS)


def _front_body(x_ref, wt_ref, csel_ref, h_ref, ac_ref):
    h = jnp.dot(x_ref[...], wt_ref[...], preferred_element_type=jnp.float32)
    h_ref[...] = h
    ac_ref[...] = jnp.dot(h, csel_ref[...], preferred_element_type=jnp.float32)


def _sc_body(row_hbm, col_hbm, ac_hbm, h_hbm, zs_hbm, zz_hbm,
             s_out_hbm, z_out_hbm,
             s_sp, z_sp, ridx, cidx, arow, brow, hrows, pbuf, sem):
    cid = lax.axis_index("c")
    sid = lax.axis_index("s")
    wid = sid * NC + cid

    pltpu.sync_copy(zs_hbm.at[pl.ds(sid * RPT, RPT)],
                    s_sp.at[pl.ds(sid * RPT, RPT)])
    pltpu.sync_copy(zz_hbm.at[pl.ds(sid * RPT, RPT)],
                    z_sp.at[pl.ds(sid * RPT, RPT)])
    plsc.subcore_barrier()

    def chunk_body(t, carry):
        base = (wid * T_PER_W + t) * K
        pltpu.sync_copy(row_hbm.at[pl.ds(base, K)], ridx)
        pltpu.sync_copy(col_hbm.at[pl.ds(base, K)], cidx)
        ga = pltpu.async_copy(ac_hbm.at[ridx], arow, sem)
        gb = pltpu.async_copy(ac_hbm.at[cidx], brow, sem)
        gh = pltpu.async_copy(h_hbm.at[cidx], hrows, sem)
        ga.wait()
        gb.wait()
        gh.wait()

        def edge_body(i, c2):
            s = arow[i, :] + _rot8(brow[i, :])
            p = jnp.exp(jnp.maximum(s, 0.2 * s))
            pbuf[i, :] = p
            for j in range(H):
                msg = p[j] * hrows[i, pl.ds(j * DH, DH)]
                hrows[i, pl.ds(j * DH, DH)] = msg
            return c2

        lax.fori_loop(0, K, edge_body, None)
        pltpu.sync_copy(hrows, s_sp.at[ridx], add=True)
        pltpu.sync_copy(pbuf, z_sp.at[ridx], add=True)
        return carry

    lax.fori_loop(0, T_PER_W, chunk_body, None)
    plsc.subcore_barrier()
    pltpu.sync_copy(s_sp.at[pl.ds(sid * RPT, RPT)],
                    s_out_hbm.at[pl.ds(cid * NPAD + sid * RPT, RPT)])
    pltpu.sync_copy(z_sp.at[pl.ds(sid * RPT, RPT)],
                    z_out_hbm.at[pl.ds(cid * NPAD + sid * RPT, RPT)])


def _combine_body(m0_ref, m1_ref, z0_ref, z1_ref, psel_ref, out_ref):
    msg = m0_ref[...] + m1_ref[...]
    z = z0_ref[...] + z1_ref[...]
    zrep = jnp.dot(z, psel_ref[...], preferred_element_type=jnp.float32)
    out_ref[...] = msg / zrep


def kernel(x, edge_indices, W, src_attn, dst_attn):
    xp = jnp.zeros((NPAD, D), jnp.float32).at[:N].set(x)
    wt = W.T
    sel8 = (jnp.arange(D)[:, None] // DH == jnp.arange(H)[None, :])
    sel8 = sel8.astype(jnp.float32)
    asel = sel8 * src_attn.reshape(D)[:, None]
    dsel = sel8 * dst_attn.reshape(D)[:, None]
    csel = jnp.concatenate([asel, dsel], axis=1)

    h, acomb = pl.pallas_call(
        _front_body,
        grid=(NPAD // FBLK,),
        in_specs=[
            pl.BlockSpec((FBLK, D), lambda i: (i, 0)),
            pl.BlockSpec((D, D), lambda i: (0, 0)),
            pl.BlockSpec((D, 16), lambda i: (0, 0)),
        ],
        out_specs=[
            pl.BlockSpec((FBLK, D), lambda i: (i, 0)),
            pl.BlockSpec((FBLK, 16), lambda i: (i, 0)),
        ],
        out_shape=[
            jax.ShapeDtypeStruct((NPAD, D), jnp.float32),
            jax.ShapeDtypeStruct((NPAD, 16), jnp.float32),
        ],
    )(xp, wt, csel)

    loops = jnp.arange(N, dtype=jnp.int32)
    pad = jnp.full((E_PAD - E - N,), N, dtype=jnp.int32)
    row = jnp.concatenate([edge_indices[0], loops, pad])
    col = jnp.concatenate([edge_indices[1], loops, pad])
    zeros_s = jnp.zeros((NPAD, D), jnp.float32)
    zeros_z = jnp.zeros((NPAD, 16), jnp.float32)

    mesh = plsc.VectorSubcoreMesh(core_axis_name="c", subcore_axis_name="s",
                                  num_cores=NC, num_subcores=NS)
    s_out, z_out = pl.kernel(
        _sc_body,
        out_type=[
            jax.ShapeDtypeStruct((NC * NPAD, D), jnp.float32),
            jax.ShapeDtypeStruct((NC * NPAD, 16), jnp.float32),
        ],
        mesh=mesh,
        compiler_params=pltpu.CompilerParams(use_tc_tiling_on_sc=False),
        scratch_types=[
            pltpu.VMEM_SHARED((NPAD, D), jnp.float32),
            pltpu.VMEM_SHARED((NPAD, 16), jnp.float32),
            pltpu.VMEM((K,), jnp.int32),
            pltpu.VMEM((K,), jnp.int32),
            pltpu.VMEM((K, 16), jnp.float32),
            pltpu.VMEM((K, 16), jnp.float32),
            pltpu.VMEM((K, D), jnp.float32),
            pltpu.VMEM((K, 16), jnp.float32),
            pltpu.SemaphoreType.DMA,
        ],
    )(row, col, acomb, h, zeros_s, zeros_z)

    psel = (jnp.arange(16)[:, None] == jnp.arange(D)[None, :] // DH)
    psel = psel.astype(jnp.float32)

    out = pl.pallas_call(
        _combine_body,
        grid=(N // CBLK,),
        in_specs=[
            pl.BlockSpec((CBLK, D), lambda i: (i, 0)),
            pl.BlockSpec((CBLK, D), lambda i: (i, 0)),
            pl.BlockSpec((CBLK, 16), lambda i: (i, 0)),
            pl.BlockSpec((CBLK, 16), lambda i: (i, 0)),
            pl.BlockSpec((16, D), lambda i: (0, 0)),
        ],
        out_specs=pl.BlockSpec((CBLK, D), lambda i: (i, 0)),
        out_shape=jax.ShapeDtypeStruct((N, D), jnp.float32),
    )(s_out[:NPAD], s_out[NPAD:], z_out[:NPAD], z_out[NPAD:], psel)

    return out


# speedup vs baseline: 81.0683x; 1.5009x over previous
import jax
import jax.numpy as jnp
from jax import lax
from jax.experimental import pallas as pl
from jax.experimental.pallas import tpu as pltpu, tpu_sc as plsc

N = 10000
E = 320000
D = 128
H = 8
DH = 16

NPAD = 10240
ROWW = D + 16
NC = 2
NS = 16
NW = NC * NS
K = 96
T_PER_W = 108
E_PAD = NW * K * T_PER_W
RPT = NPAD // NS
FBLK = 1024
CBLK = 1000

_GDN = lax.GatherDimensionNumbers(
    offset_dims=(), collapsed_slice_dims=(0,), start_index_map=(0,))


def _rot8(v):
    shift = jnp.bitwise_and(lax.iota(jnp.int32, 16) + 8, 15)
    return lax.gather(v, shift[:, None], _GDN, (1,),
                      mode=lax.GatherScatterMode.PROMISE_IN_BOUNDS)


def _front_body(x_ref, wt_ref, csel_ref, hx_ref, ac_ref):
    h = jnp.dot(x_ref[...], wt_ref[...], preferred_element_type=jnp.float32)
    ac = jnp.dot(h, csel_ref[...], preferred_element_type=jnp.float32)
    hx_ref[...] = jnp.concatenate([h, ac], axis=1)
    ac_ref[...] = ac


def _sc_body(row_hbm, col_hbm, hx_hbm, ac_hbm, zs_hbm, s_out_hbm,
             s_sp, ridx_a, cidx_a, arow_a, hx_a, ridx_b, cidx_b, arow_b,
             hx_b, sem_a, sem_b):
    cid = lax.axis_index("c")
    sid = lax.axis_index("s")
    wid = sid * NC + cid

    pltpu.sync_copy(zs_hbm.at[pl.ds(sid * RPT, RPT)],
                    s_sp.at[pl.ds(sid * RPT, RPT)])
    plsc.subcore_barrier()

    def issue(t, ridx, cidx, arow, hx, sem):
        base = (wid * T_PER_W + t) * K
        pltpu.sync_copy(row_hbm.at[pl.ds(base, K)], ridx)
        pltpu.sync_copy(col_hbm.at[pl.ds(base, K)], cidx)
        pltpu.async_copy(ac_hbm.at[ridx], arow, sem)
        pltpu.async_copy(hx_hbm.at[cidx], hx, sem)

    def process(ridx, cidx, arow, hx, sem):
        pltpu.make_async_copy(ac_hbm.at[ridx], arow, sem).wait()
        pltpu.make_async_copy(hx_hbm.at[cidx], hx, sem).wait()

        def edge_body(i, c2):
            s = arow[i, :] + _rot8(hx[i, pl.ds(D, 16)])
            p = jnp.exp(jnp.maximum(s, 0.2 * s))
            hx[i, pl.ds(D, 16)] = p
            for j in range(H):
                msg = p[j] * hx[i, pl.ds(j * DH, DH)]
                hx[i, pl.ds(j * DH, DH)] = msg
            return c2

        lax.fori_loop(0, K, edge_body, None)
        pltpu.sync_copy(hx, s_sp.at[ridx], add=True)

    issue(0, ridx_a, cidx_a, arow_a, hx_a, sem_a)

    def pipe_body(i, carry):
        t0 = 2 * i
        issue(t0 + 1, ridx_b, cidx_b, arow_b, hx_b, sem_b)
        process(ridx_a, cidx_a, arow_a, hx_a, sem_a)

        @pl.when(i < T_PER_W // 2 - 1)
        def _():
            issue(t0 + 2, ridx_a, cidx_a, arow_a, hx_a, sem_a)

        process(ridx_b, cidx_b, arow_b, hx_b, sem_b)
        return carry

    lax.fori_loop(0, T_PER_W // 2, pipe_body, None)
    plsc.subcore_barrier()
    pltpu.sync_copy(s_sp.at[pl.ds(sid * RPT, RPT)],
                    s_out_hbm.at[pl.ds(cid * NPAD + sid * RPT, RPT)])


def _combine_body(s0_ref, s1_ref, msel_ref, zsel_ref, out_ref):
    t = s0_ref[...] + s1_ref[...]
    msg = jnp.dot(t, msel_ref[...], preferred_element_type=jnp.float32)
    zrep = jnp.dot(t, zsel_ref[...], preferred_element_type=jnp.float32)
    out_ref[...] = msg / zrep


def kernel(x, edge_indices, W, src_attn, dst_attn):
    xp = jnp.zeros((NPAD, D), jnp.float32).at[:N].set(x)
    wt = W.T
    sel8 = (jnp.arange(D)[:, None] // DH == jnp.arange(H)[None, :])
    sel8 = sel8.astype(jnp.float32)
    asel = sel8 * src_attn.reshape(D)[:, None]
    dsel = sel8 * dst_attn.reshape(D)[:, None]
    csel = jnp.concatenate([asel, dsel], axis=1)

    hx, acomb = pl.pallas_call(
        _front_body,
        grid=(NPAD // FBLK,),
        in_specs=[
            pl.BlockSpec((FBLK, D), lambda i: (i, 0)),
            pl.BlockSpec((D, D), lambda i: (0, 0)),
            pl.BlockSpec((D, 16), lambda i: (0, 0)),
        ],
        out_specs=[
            pl.BlockSpec((FBLK, ROWW), lambda i: (i, 0)),
            pl.BlockSpec((FBLK, 16), lambda i: (i, 0)),
        ],
        out_shape=[
            jax.ShapeDtypeStruct((NPAD, ROWW), jnp.float32),
            jax.ShapeDtypeStruct((NPAD, 16), jnp.float32),
        ],
    )(xp, wt, csel)

    loops = jnp.arange(N, dtype=jnp.int32)
    pad = jnp.full((E_PAD - E - N,), N, dtype=jnp.int32)
    row = jnp.concatenate([edge_indices[0], loops, pad])
    col = jnp.concatenate([edge_indices[1], loops, pad])
    zeros_s = jnp.zeros((NPAD, ROWW), jnp.float32)

    mesh = plsc.VectorSubcoreMesh(core_axis_name="c", subcore_axis_name="s",
                                  num_cores=NC, num_subcores=NS)
    s_out = pl.kernel(
        _sc_body,
        out_type=jax.ShapeDtypeStruct((NC * NPAD, ROWW), jnp.float32),
        mesh=mesh,
        compiler_params=pltpu.CompilerParams(use_tc_tiling_on_sc=False),
        scratch_types=[
            pltpu.VMEM_SHARED((NPAD, ROWW), jnp.float32),
            pltpu.VMEM((K,), jnp.int32),
            pltpu.VMEM((K,), jnp.int32),
            pltpu.VMEM((K, 16), jnp.float32),
            pltpu.VMEM((K, ROWW), jnp.float32),
            pltpu.VMEM((K,), jnp.int32),
            pltpu.VMEM((K,), jnp.int32),
            pltpu.VMEM((K, 16), jnp.float32),
            pltpu.VMEM((K, ROWW), jnp.float32),
            pltpu.SemaphoreType.DMA,
            pltpu.SemaphoreType.DMA,
        ],
    )(row, col, hx, acomb, zeros_s)

    msel = jnp.zeros((ROWW, D), jnp.float32).at[:D, :].set(jnp.eye(D))
    zsel = jnp.zeros((ROWW, D), jnp.float32).at[D:, :].set(
        (jnp.arange(16)[:, None] == jnp.arange(D)[None, :] // DH)
        .astype(jnp.float32))

    out = pl.pallas_call(
        _combine_body,
        grid=(N // CBLK,),
        in_specs=[
            pl.BlockSpec((CBLK, ROWW), lambda i: (i, 0)),
            pl.BlockSpec((CBLK, ROWW), lambda i: (i, 0)),
            pl.BlockSpec((ROWW, D), lambda i: (0, 0)),
            pl.BlockSpec((ROWW, D), lambda i: (0, 0)),
        ],
        out_specs=pl.BlockSpec((CBLK, D), lambda i: (i, 0)),
        out_shape=jax.ShapeDtypeStruct((N, D), jnp.float32),
    )(s_out[:NPAD], s_out[NPAD:], msel, zsel)

    return out


# speedup vs baseline: 97.6131x; 1.2041x over previous
import jax
import jax.numpy as jnp
from jax import lax
from jax.experimental import pallas as pl
from jax.experimental.pallas import tpu as pltpu, tpu_sc as plsc

N = 10000
E = 320000
D = 128
H = 8
DH = 16

NPAD = 10240
ROWW = D + 16
NC = 2
NS = 16
NW = NC * NS
K = 64
T_PER_W = 162
SUP = 18
NSB = T_PER_W // SUP
E_PAD = NW * K * T_PER_W
RPT = NPAD // NS
FBLK = 1024
CBLK = 1000

_GDN = lax.GatherDimensionNumbers(
    offset_dims=(), collapsed_slice_dims=(0,), start_index_map=(0,))


def _rot8(v):
    shift = jnp.bitwise_and(lax.iota(jnp.int32, 16) + 8, 15)
    return lax.gather(v, shift[:, None], _GDN, (1,),
                      mode=lax.GatherScatterMode.PROMISE_IN_BOUNDS)


def _front_body(x_ref, wt_ref, csel_ref, hx_ref, ac_ref):
    h = jnp.dot(x_ref[...], wt_ref[...], preferred_element_type=jnp.float32)
    ac = jnp.dot(h, csel_ref[...], preferred_element_type=jnp.float32)
    hx_ref[...] = jnp.concatenate([h, ac], axis=1)
    ac_ref[...] = ac


def _sc_body(row2_hbm, col2_hbm, hx_hbm, ac_hbm, zs_hbm, s_out_hbm,
             s_sp, ridx, cidx, ar0, ar1, ar2, hx0, hx1, hx2,
             g0, g1, g2, s0, s1, s2):
    cid = lax.axis_index("c")
    sid = lax.axis_index("s")
    wid = sid * NC + cid
    ars = (ar0, ar1, ar2)
    hxs = (hx0, hx1, hx2)
    gsems = (g0, g1, g2)
    ssems = (s0, s1, s2)

    pltpu.sync_copy(zs_hbm.at[pl.ds(sid * RPT, RPT)],
                    s_sp.at[pl.ds(sid * RPT, RPT)])
    plsc.subcore_barrier()

    def gissue(u, b):
        pltpu.async_copy(ac_hbm.at[ridx.at[u]], ars[b], gsems[b])
        pltpu.async_copy(hx_hbm.at[cidx.at[u]], hxs[b], gsems[b])

    def gwait(b):
        pltpu.make_async_copy(ac_hbm.at[ridx.at[0]], ars[b], gsems[b]).wait()
        pltpu.make_async_copy(hx_hbm.at[cidx.at[0]], hxs[b], gsems[b]).wait()

    def swait(b):
        pltpu.make_async_copy(hxs[b], s_sp.at[ridx.at[0]], ssems[b]).wait()

    def compute(b):
        arow = ars[b]
        hx = hxs[b]

        def edge_body(i, c2):
            s = arow[i, :] + _rot8(hx[i, pl.ds(D, 16)])
            p = jnp.exp(jnp.maximum(s, 0.2 * s))
            hx[i, pl.ds(D, 16)] = p
            for j in range(H):
                msg = p[j] * hx[i, pl.ds(j * DH, DH)]
                hx[i, pl.ds(j * DH, DH)] = msg
            return c2

        lax.fori_loop(0, K, edge_body, None)

    def sb_body(sb, carry):
        rbase = wid * T_PER_W + sb * SUP

        @pl.when(sb > 0)
        def _():
            swait(0)
            swait(1)
            swait(2)

        pltpu.sync_copy(row2_hbm.at[pl.ds(rbase, SUP)], ridx)
        pltpu.sync_copy(col2_hbm.at[pl.ds(rbase, SUP)], cidx)
        gissue(0, 0)
        gissue(1, 1)
        for u in range(SUP):
            b = u % 3
            gwait(b)
            compute(b)
            if u + 2 < SUP:
                b2 = (u + 2) % 3
                if u >= 1:
                    swait(b2)
                gissue(u + 2, b2)
            pltpu.async_copy(hxs[b], s_sp.at[ridx.at[u]], ssems[b], add=True)
        return carry

    lax.fori_loop(0, NSB, sb_body, None)
    swait(0)
    swait(1)
    swait(2)
    plsc.subcore_barrier()
    pltpu.sync_copy(s_sp.at[pl.ds(sid * RPT, RPT)],
                    s_out_hbm.at[pl.ds(cid * NPAD + sid * RPT, RPT)])


def _combine_body(s0_ref, s1_ref, msel_ref, zsel_ref, out_ref):
    t = s0_ref[...] + s1_ref[...]
    msg = jnp.dot(t, msel_ref[...], preferred_element_type=jnp.float32)
    zrep = jnp.dot(t, zsel_ref[...], preferred_element_type=jnp.float32)
    out_ref[...] = msg / zrep


def kernel(x, edge_indices, W, src_attn, dst_attn):
    xp = jnp.zeros((NPAD, D), jnp.float32).at[:N].set(x)
    wt = W.T
    sel8 = (jnp.arange(D)[:, None] // DH == jnp.arange(H)[None, :])
    sel8 = sel8.astype(jnp.float32)
    asel = sel8 * src_attn.reshape(D)[:, None]
    dsel = sel8 * dst_attn.reshape(D)[:, None]
    csel = jnp.concatenate([asel, dsel], axis=1)

    hx, acomb = pl.pallas_call(
        _front_body,
        grid=(NPAD // FBLK,),
        in_specs=[
            pl.BlockSpec((FBLK, D), lambda i: (i, 0)),
            pl.BlockSpec((D, D), lambda i: (0, 0)),
            pl.BlockSpec((D, 16), lambda i: (0, 0)),
        ],
        out_specs=[
            pl.BlockSpec((FBLK, ROWW), lambda i: (i, 0)),
            pl.BlockSpec((FBLK, 16), lambda i: (i, 0)),
        ],
        out_shape=[
            jax.ShapeDtypeStruct((NPAD, ROWW), jnp.float32),
            jax.ShapeDtypeStruct((NPAD, 16), jnp.float32),
        ],
    )(xp, wt, csel)

    loops = jnp.arange(N, dtype=jnp.int32)
    pad = jnp.full((E_PAD - E - N,), N, dtype=jnp.int32)
    row = jnp.concatenate([edge_indices[0], loops, pad]).reshape(E_PAD // K, K)
    col = jnp.concatenate([edge_indices[1], loops, pad]).reshape(E_PAD // K, K)
    zeros_s = jnp.zeros((NPAD, ROWW), jnp.float32)

    mesh = plsc.VectorSubcoreMesh(core_axis_name="c", subcore_axis_name="s",
                                  num_cores=NC, num_subcores=NS)
    s_out = pl.kernel(
        _sc_body,
        out_type=jax.ShapeDtypeStruct((NC * NPAD, ROWW), jnp.float32),
        mesh=mesh,
        compiler_params=pltpu.CompilerParams(use_tc_tiling_on_sc=False),
        scratch_types=[
            pltpu.VMEM_SHARED((NPAD, ROWW), jnp.float32),
            pltpu.VMEM((SUP, K), jnp.int32),
            pltpu.VMEM((SUP, K), jnp.int32),
            pltpu.VMEM((K, 16), jnp.float32),
            pltpu.VMEM((K, 16), jnp.float32),
            pltpu.VMEM((K, 16), jnp.float32),
            pltpu.VMEM((K, ROWW), jnp.float32),
            pltpu.VMEM((K, ROWW), jnp.float32),
            pltpu.VMEM((K, ROWW), jnp.float32),
            pltpu.SemaphoreType.DMA,
            pltpu.SemaphoreType.DMA,
            pltpu.SemaphoreType.DMA,
            pltpu.SemaphoreType.DMA,
            pltpu.SemaphoreType.DMA,
            pltpu.SemaphoreType.DMA,
        ],
    )(row, col, hx, acomb, zeros_s)

    msel = jnp.zeros((ROWW, D), jnp.float32).at[:D, :].set(jnp.eye(D))
    zsel = jnp.zeros((ROWW, D), jnp.float32).at[D:, :].set(
        (jnp.arange(16)[:, None] == jnp.arange(D)[None, :] // DH)
        .astype(jnp.float32))

    out = pl.pallas_call(
        _combine_body,
        grid=(N // CBLK,),
        in_specs=[
            pl.BlockSpec((CBLK, ROWW), lambda i: (i, 0)),
            pl.BlockSpec((CBLK, ROWW), lambda i: (i, 0)),
            pl.BlockSpec((ROWW, D), lambda i: (0, 0)),
            pl.BlockSpec((ROWW, D), lambda i: (0, 0)),
        ],
        out_specs=pl.BlockSpec((CBLK, D), lambda i: (i, 0)),
        out_shape=jax.ShapeDtypeStruct((N, D), jnp.float32),
    )(s_out[:NPAD], s_out[NPAD:], msel, zsel)

    return out


# speedup vs baseline: 103.0987x; 1.0562x over previous
import jax
import jax.numpy as jnp
from jax import lax
from jax.experimental import pallas as pl
from jax.experimental.pallas import tpu as pltpu, tpu_sc as plsc

N = 10000
E = 320000
D = 128
H = 8
DH = 16

NPAD = 10240
ROWW = D + 16
NC = 2
NS = 16
NW = NC * NS
K = 64
T_PER_W = 162
SUP = 18
NSB = T_PER_W // SUP
E_PAD = NW * K * T_PER_W
RPT = NPAD // NS
FBLK = 1024
CBLK = 1000

_GDN = lax.GatherDimensionNumbers(
    offset_dims=(), collapsed_slice_dims=(0,), start_index_map=(0,))


def _rot8(v):
    shift = jnp.bitwise_and(lax.iota(jnp.int32, 16) + 8, 15)
    return lax.gather(v, shift[:, None], _GDN, (1,),
                      mode=lax.GatherScatterMode.PROMISE_IN_BOUNDS)


def _front_body(x_ref, wt_ref, csel_ref, hx_ref, ac_ref):
    h = jnp.dot(x_ref[...], wt_ref[...], preferred_element_type=jnp.float32)
    ac = jnp.dot(h, csel_ref[...], preferred_element_type=jnp.float32)
    hx_ref[...] = jnp.concatenate([h, ac], axis=1)
    ac_ref[...] = ac


def _sc_body(row2_hbm, col2_hbm, hx_hbm, ac_hbm, zs_hbm, m_out_hbm, z_out_hbm,
             s_sp, ridx, cidx, ar0, ar1, ar2, hx0, hx1, hx2,
             g0, g1, g2, s0, s1, s2):
    cid = lax.axis_index("c")
    sid = lax.axis_index("s")
    wid = sid * NC + cid
    ars = (ar0, ar1, ar2)
    hxs = (hx0, hx1, hx2)
    gsems = (g0, g1, g2)
    ssems = (s0, s1, s2)

    pltpu.sync_copy(zs_hbm.at[pl.ds(sid * RPT, RPT)],
                    s_sp.at[pl.ds(sid * RPT, RPT)])
    plsc.subcore_barrier()

    def gissue(u, b):
        pltpu.async_copy(ac_hbm.at[ridx.at[u]], ars[b], gsems[b])
        pltpu.async_copy(hx_hbm.at[cidx.at[u]], hxs[b], gsems[b])

    def gwait(b):
        pltpu.make_async_copy(ac_hbm.at[ridx.at[0]], ars[b], gsems[b]).wait()
        pltpu.make_async_copy(hx_hbm.at[cidx.at[0]], hxs[b], gsems[b]).wait()

    def swait(b):
        pltpu.make_async_copy(hxs[b], s_sp.at[ridx.at[0]], ssems[b]).wait()

    def compute(b):
        arow = ars[b]
        hx = hxs[b]

        def edge_body(i2, c2):
            for i in (2 * i2, 2 * i2 + 1):
                s = arow[i, :] + _rot8(hx[i, pl.ds(D, 16)])
                p = jnp.exp(jnp.maximum(s, 0.2 * s))
                hx[i, pl.ds(D, 16)] = p
                for j in range(H):
                    msg = p[j] * hx[i, pl.ds(j * DH, DH)]
                    hx[i, pl.ds(j * DH, DH)] = msg
            return c2

        lax.fori_loop(0, K // 2, edge_body, None)

    def sb_body(sb, carry):
        rbase = wid * T_PER_W + sb * SUP

        @pl.when(sb > 0)
        def _():
            swait(0)
            swait(1)
            swait(2)

        pltpu.sync_copy(row2_hbm.at[pl.ds(rbase, SUP)], ridx)
        pltpu.sync_copy(col2_hbm.at[pl.ds(rbase, SUP)], cidx)
        gissue(0, 0)
        gissue(1, 1)
        for u in range(SUP):
            b = u % 3
            gwait(b)
            compute(b)
            if u + 2 < SUP:
                b2 = (u + 2) % 3
                if u >= 1:
                    swait(b2)
                gissue(u + 2, b2)
            pltpu.async_copy(hxs[b], s_sp.at[ridx.at[u]], ssems[b], add=True)
        return carry

    lax.fori_loop(0, NSB, sb_body, None)
    swait(0)
    swait(1)
    swait(2)
    plsc.subcore_barrier()
    pltpu.sync_copy(s_sp.at[pl.ds(sid * RPT, RPT), pl.ds(0, D)],
                    m_out_hbm.at[pl.ds(cid * NPAD + sid * RPT, RPT)])
    pltpu.sync_copy(s_sp.at[pl.ds(sid * RPT, RPT), pl.ds(D, 16)],
                    z_out_hbm.at[pl.ds(cid * NPAD + sid * RPT, RPT)])


def _combine_body(m0_ref, m1_ref, z0_ref, z1_ref, psel_ref, out_ref):
    msg = m0_ref[...] + m1_ref[...]
    z = z0_ref[...] + z1_ref[...]
    zrep = jnp.dot(z, psel_ref[...], preferred_element_type=jnp.float32)
    out_ref[...] = msg / zrep


def kernel(x, edge_indices, W, src_attn, dst_attn):
    xp = jnp.zeros((NPAD, D), jnp.float32).at[:N].set(x)
    wt = W.T
    sel8 = (jnp.arange(D)[:, None] // DH == jnp.arange(H)[None, :])
    sel8 = sel8.astype(jnp.float32)
    asel = sel8 * src_attn.reshape(D)[:, None]
    dsel = sel8 * dst_attn.reshape(D)[:, None]
    csel = jnp.concatenate([asel, dsel], axis=1)

    hx, acomb = pl.pallas_call(
        _front_body,
        grid=(NPAD // FBLK,),
        in_specs=[
            pl.BlockSpec((FBLK, D), lambda i: (i, 0)),
            pl.BlockSpec((D, D), lambda i: (0, 0)),
            pl.BlockSpec((D, 16), lambda i: (0, 0)),
        ],
        out_specs=[
            pl.BlockSpec((FBLK, ROWW), lambda i: (i, 0)),
            pl.BlockSpec((FBLK, 16), lambda i: (i, 0)),
        ],
        out_shape=[
            jax.ShapeDtypeStruct((NPAD, ROWW), jnp.float32),
            jax.ShapeDtypeStruct((NPAD, 16), jnp.float32),
        ],
    )(xp, wt, csel)

    loops = jnp.arange(N, dtype=jnp.int32)
    pad = jnp.full((E_PAD - E - N,), N, dtype=jnp.int32)
    row = jnp.concatenate([edge_indices[0], loops, pad]).reshape(E_PAD // K, K)
    col = jnp.concatenate([edge_indices[1], loops, pad]).reshape(E_PAD // K, K)
    zeros_s = jnp.zeros((NPAD, ROWW), jnp.float32)

    mesh = plsc.VectorSubcoreMesh(core_axis_name="c", subcore_axis_name="s",
                                  num_cores=NC, num_subcores=NS)
    m_out, z_out = pl.kernel(
        _sc_body,
        out_type=[
            jax.ShapeDtypeStruct((NC * NPAD, D), jnp.float32),
            jax.ShapeDtypeStruct((NC * NPAD, 16), jnp.float32),
        ],
        mesh=mesh,
        compiler_params=pltpu.CompilerParams(use_tc_tiling_on_sc=False),
        scratch_types=[
            pltpu.VMEM_SHARED((NPAD, ROWW), jnp.float32),
            pltpu.VMEM((SUP, K), jnp.int32),
            pltpu.VMEM((SUP, K), jnp.int32),
            pltpu.VMEM((K, 16), jnp.float32),
            pltpu.VMEM((K, 16), jnp.float32),
            pltpu.VMEM((K, 16), jnp.float32),
            pltpu.VMEM((K, ROWW), jnp.float32),
            pltpu.VMEM((K, ROWW), jnp.float32),
            pltpu.VMEM((K, ROWW), jnp.float32),
            pltpu.SemaphoreType.DMA,
            pltpu.SemaphoreType.DMA,
            pltpu.SemaphoreType.DMA,
            pltpu.SemaphoreType.DMA,
            pltpu.SemaphoreType.DMA,
            pltpu.SemaphoreType.DMA,
        ],
    )(row, col, hx, acomb, zeros_s)

    psel = (jnp.arange(16)[:, None] == jnp.arange(D)[None, :] // DH)
    psel = psel.astype(jnp.float32)

    out = pl.pallas_call(
        _combine_body,
        grid=(N // CBLK,),
        in_specs=[
            pl.BlockSpec((CBLK, D), lambda i: (i, 0)),
            pl.BlockSpec((CBLK, D), lambda i: (i, 0)),
            pl.BlockSpec((CBLK, 16), lambda i: (i, 0)),
            pl.BlockSpec((CBLK, 16), lambda i: (i, 0)),
            pl.BlockSpec((16, D), lambda i: (0, 0)),
        ],
        out_specs=pl.BlockSpec((CBLK, D), lambda i: (i, 0)),
        out_shape=jax.ShapeDtypeStruct((N, D), jnp.float32),
    )(m_out[:NPAD], m_out[NPAD:], z_out[:NPAD], z_out[NPAD:], psel)

    return out


# speedup vs baseline: 108.4019x; 1.0514x over previous
import jax
import jax.numpy as jnp
from jax import lax
from jax.experimental import pallas as pl
from jax.experimental.pallas import tpu as pltpu, tpu_sc as plsc

N = 10000
E = 320000
D = 128
H = 8
DH = 16

NPAD = 10240
ROWW = D + 16
NC = 2
NS = 16
NW = NC * NS
K = 64
T_PER_W = 162
SUP = 18
NSB = T_PER_W // SUP
E_PAD = NW * K * T_PER_W
RPT = NPAD // NS
FBLK = 1024
CBLK = 1000

_GDN = lax.GatherDimensionNumbers(
    offset_dims=(), collapsed_slice_dims=(0,), start_index_map=(0,))


def _rot8(v):
    shift = jnp.bitwise_and(lax.iota(jnp.int32, 16) + 8, 15)
    return lax.gather(v, shift[:, None], _GDN, (1,),
                      mode=lax.GatherScatterMode.PROMISE_IN_BOUNDS)


def _front_body(x_ref, wt_ref, csel_ref, hx_ref, ac_ref):
    h = jnp.dot(x_ref[...], wt_ref[...], preferred_element_type=jnp.float32)
    ac = jnp.dot(h, csel_ref[...], preferred_element_type=jnp.float32)
    hx_ref[...] = jnp.concatenate([h, ac], axis=1)
    ac_ref[...] = ac


def _sc_body(row2_hbm, col2_hbm, hx_hbm, ac_hbm, zs_hbm, m_out_hbm, z_out_hbm,
             s_sp, ridx, cidx, ar0, ar1, ar2, hx0, hx1, hx2,
             g0, g1, g2, s0, s1, s2):
    cid = lax.axis_index("c")
    sid = lax.axis_index("s")
    wid = sid * NC + cid
    ars = (ar0, ar1, ar2)
    hxs = (hx0, hx1, hx2)
    gsems = (g0, g1, g2)
    ssems = (s0, s1, s2)

    pltpu.sync_copy(zs_hbm.at[pl.ds(sid * RPT, RPT)],
                    s_sp.at[pl.ds(sid * RPT, RPT)])
    plsc.subcore_barrier()

    def gissue(u, b):
        pltpu.async_copy(ac_hbm.at[ridx.at[u]], ars[b], gsems[b])
        pltpu.async_copy(hx_hbm.at[cidx.at[u]], hxs[b], gsems[b])

    def gwait(b):
        pltpu.make_async_copy(ac_hbm.at[ridx.at[0]], ars[b], gsems[b]).wait()
        pltpu.make_async_copy(hx_hbm.at[cidx.at[0]], hxs[b], gsems[b]).wait()

    def swait(b):
        pltpu.make_async_copy(hxs[b], s_sp.at[ridx.at[0]], ssems[b]).wait()

    def compute(b):
        arow = ars[b]
        hx = hxs[b]

        def edge_body(i2, c2):
            for i in (2 * i2, 2 * i2 + 1):
                s = arow[i, :] + _rot8(hx[i, pl.ds(D, 16)])
                p = jnp.exp(jnp.maximum(s, 0.2 * s))
                hx[i, pl.ds(D, 16)] = p
                for j in range(H):
                    msg = p[j] * hx[i, pl.ds(j * DH, DH)]
                    hx[i, pl.ds(j * DH, DH)] = msg
            return c2

        lax.fori_loop(0, K // 2, edge_body, None)

    def sb_body(sb, carry):
        rbase = wid * T_PER_W + sb * SUP

        @pl.when(sb > 0)
        def _():
            swait(0)
            swait(1)
            swait(2)

        pltpu.sync_copy(row2_hbm.at[pl.ds(rbase, SUP)], ridx)
        pltpu.sync_copy(col2_hbm.at[pl.ds(rbase, SUP)], cidx)

        def synth(u):
            tg = rbase + u

            @pl.when(tg >= E // K)
            def _():
                nbase = (tg - E // K) * K
                for g in range(K // 16):
                    v = jnp.minimum(nbase + g * 16 + lax.iota(jnp.int32, 16),
                                    N)
                    ridx[u, pl.ds(g * 16, 16)] = v
                    cidx[u, pl.ds(g * 16, 16)] = v

        synth(0)
        synth(1)
        gissue(0, 0)
        gissue(1, 1)
        for u in range(SUP):
            b = u % 3
            gwait(b)
            compute(b)
            if u + 2 < SUP:
                b2 = (u + 2) % 3
                if u >= 1:
                    swait(b2)
                synth(u + 2)
                gissue(u + 2, b2)
            pltpu.async_copy(hxs[b], s_sp.at[ridx.at[u]], ssems[b], add=True)
        return carry

    lax.fori_loop(0, NSB, sb_body, None)
    swait(0)
    swait(1)
    swait(2)
    plsc.subcore_barrier()
    pltpu.sync_copy(s_sp.at[pl.ds(sid * RPT, RPT), pl.ds(0, D)],
                    m_out_hbm.at[pl.ds(cid * NPAD + sid * RPT, RPT)])
    pltpu.sync_copy(s_sp.at[pl.ds(sid * RPT, RPT), pl.ds(D, 16)],
                    z_out_hbm.at[pl.ds(cid * NPAD + sid * RPT, RPT)])


def _combine_body(m0_ref, m1_ref, z0_ref, z1_ref, psel_ref, out_ref):
    msg = m0_ref[...] + m1_ref[...]
    z = z0_ref[...] + z1_ref[...]
    zrep = jnp.dot(z, psel_ref[...], preferred_element_type=jnp.float32)
    out_ref[...] = msg / zrep


def kernel(x, edge_indices, W, src_attn, dst_attn):
    xp = jnp.zeros((NPAD, D), jnp.float32).at[:N].set(x)
    wt = W.T
    sel8 = (jnp.arange(D)[:, None] // DH == jnp.arange(H)[None, :])
    sel8 = sel8.astype(jnp.float32)
    asel = sel8 * src_attn.reshape(D)[:, None]
    dsel = sel8 * dst_attn.reshape(D)[:, None]
    csel = jnp.concatenate([asel, dsel], axis=1)

    hx, acomb = pl.pallas_call(
        _front_body,
        grid=(NPAD // FBLK,),
        in_specs=[
            pl.BlockSpec((FBLK, D), lambda i: (i, 0)),
            pl.BlockSpec((D, D), lambda i: (0, 0)),
            pl.BlockSpec((D, 16), lambda i: (0, 0)),
        ],
        out_specs=[
            pl.BlockSpec((FBLK, ROWW), lambda i: (i, 0)),
            pl.BlockSpec((FBLK, 16), lambda i: (i, 0)),
        ],
        out_shape=[
            jax.ShapeDtypeStruct((NPAD, ROWW), jnp.float32),
            jax.ShapeDtypeStruct((NPAD, 16), jnp.float32),
        ],
    )(xp, wt, csel)

    ei3 = jnp.pad(edge_indices.reshape(2, E // K, K),
                  ((0, 0), (0, (E_PAD - E) // K), (0, 0)))
    row2 = ei3[0]
    col2 = ei3[1]
    zeros_s = jnp.zeros((NPAD, ROWW), jnp.float32)

    mesh = plsc.VectorSubcoreMesh(core_axis_name="c", subcore_axis_name="s",
                                  num_cores=NC, num_subcores=NS)
    m_out, z_out = pl.kernel(
        _sc_body,
        out_type=[
            jax.ShapeDtypeStruct((NC * NPAD, D), jnp.float32),
            jax.ShapeDtypeStruct((NC * NPAD, 16), jnp.float32),
        ],
        mesh=mesh,
        compiler_params=pltpu.CompilerParams(use_tc_tiling_on_sc=False),
        scratch_types=[
            pltpu.VMEM_SHARED((NPAD, ROWW), jnp.float32),
            pltpu.VMEM((SUP, K), jnp.int32),
            pltpu.VMEM((SUP, K), jnp.int32),
            pltpu.VMEM((K, 16), jnp.float32),
            pltpu.VMEM((K, 16), jnp.float32),
            pltpu.VMEM((K, 16), jnp.float32),
            pltpu.VMEM((K, ROWW), jnp.float32),
            pltpu.VMEM((K, ROWW), jnp.float32),
            pltpu.VMEM((K, ROWW), jnp.float32),
            pltpu.SemaphoreType.DMA,
            pltpu.SemaphoreType.DMA,
            pltpu.SemaphoreType.DMA,
            pltpu.SemaphoreType.DMA,
            pltpu.SemaphoreType.DMA,
            pltpu.SemaphoreType.DMA,
        ],
    )(row2, col2, hx, acomb, zeros_s)

    psel = (jnp.arange(16)[:, None] == jnp.arange(D)[None, :] // DH)
    psel = psel.astype(jnp.float32)

    out = pl.pallas_call(
        _combine_body,
        grid=(N // CBLK,),
        in_specs=[
            pl.BlockSpec((CBLK, D), lambda i: (i, 0)),
            pl.BlockSpec((CBLK, D), lambda i: (i, 0)),
            pl.BlockSpec((CBLK, 16), lambda i: (i, 0)),
            pl.BlockSpec((CBLK, 16), lambda i: (i, 0)),
            pl.BlockSpec((16, D), lambda i: (0, 0)),
        ],
        out_specs=pl.BlockSpec((CBLK, D), lambda i: (i, 0)),
        out_shape=jax.ShapeDtypeStruct((N, D), jnp.float32),
    )(m_out[:NPAD], m_out[NPAD:], z_out[:NPAD], z_out[NPAD:], psel)

    return out
